# Initial kernel scaffold; baseline (speedup 1.0000x reference)
import jax
import jax.numpy as jnp
from jax.experimental import pallas as pl


def kernel(n_feat_geo, nfeat_act, efeat_act, edge_index, W_geo1, b_geo1, W_geo2, b_geo2, W_act1, b_act1, W_act2, b_act2, W_fc, b_fc):
    raise NotImplementedError("write your pallas kernel here")


# speedup vs baseline: 3.1873x; 3.1873x over previous
import functools

import jax
import jax.numpy as jnp
from jax import lax
from jax.experimental import pallas as pl
from jax.experimental.pallas import tpu as pltpu
from jax.experimental.pallas import tpu_sc as plsc

_N = 10000
_E = 320000
_D = 128
_NC = 2
_NS = 16
_NW = _NC * _NS
_EPW = _E // _NW
_CH = 80
_NCHUNK = _EPW // _CH
_NPAD = 10112
_RPT = _NPAD // _NS
_VPR = _D // 16


def _sc_round(x, e, src3, dst3, zrows):
    with_e = e is not None
    mesh = plsc.VectorSubcoreMesh(
        core_axis_name="c", subcore_axis_name="s",
        num_cores=_NC, num_subcores=_NS)

    scratch = [
        pltpu.VMEM((_CH,), jnp.int32),
        pltpu.VMEM((_CH,), jnp.int32),
        pltpu.VMEM((_CH, _D), jnp.float32),
        pltpu.VMEM_SHARED((_NPAD, _D), jnp.float32),
        pltpu.SemaphoreType.DMA,
    ]
    if with_e:
        scratch.insert(3, pltpu.VMEM((_CH, _D), jnp.float32))

    def body(*refs):
        if with_e:
            (x_hbm, e_hbm, src_hbm, dst_hbm, z_hbm, out_hbm,
             sidx, didx, rows, ebuf, acc, sem) = refs
        else:
            (x_hbm, src_hbm, dst_hbm, z_hbm, out_hbm,
             sidx, didx, rows, acc, sem) = refs
            e_hbm = None
        c = lax.axis_index("c")
        s = lax.axis_index("s")
        wid = c * _NS + s

        pltpu.sync_copy(z_hbm, acc.at[pl.ds(s * _RPT, _RPT)])
        plsc.subcore_barrier()

        def chunk(j, carry):
            pltpu.sync_copy(src_hbm.at[wid, j], sidx)
            pltpu.sync_copy(dst_hbm.at[wid, j], didx)
            gcp = pltpu.async_copy(x_hbm.at[sidx], rows, sem)
            if with_e:
                off = wid * _EPW + j * _CH
                pltpu.sync_copy(e_hbm.at[pl.ds(off, _CH)], ebuf)
            gcp.wait()
            if with_e:
                def rloop(i, c2):
                    for jj in range(_VPR):
                        sl = pl.ds(jj * 16, 16)
                        v = rows[i, sl] + ebuf[i, sl]
                        rows[i, sl] = jnp.maximum(v, 0.0)
                    return c2
                lax.fori_loop(0, _CH, rloop, 0)
            pltpu.sync_copy(rows, acc.at[didx], add=True)
            return carry
        lax.fori_loop(0, _NCHUNK, chunk, 0)
        plsc.subcore_barrier()

        pltpu.sync_copy(acc.at[pl.ds(s * _RPT, _RPT)],
                        out_hbm.at[c, pl.ds(s * _RPT, _RPT)])

    run = pl.kernel(
        body,
        out_type=jax.ShapeDtypeStruct((_NC, _NPAD, _D), jnp.float32),
        mesh=mesh,
        scratch_types=scratch,
    )
    if with_e:
        return run(x, e, src3, dst3, zrows)
    return run(x, src3, dst3, zrows)


def _tc_layer(x, acc, W, b, slope):
    bn = 2000

    def body(x_ref, a_ref, w_ref, b_ref, o_ref):
        t = x_ref[...] + a_ref[0] + a_ref[1]
        y = jnp.dot(t, w_ref[...], preferred_element_type=jnp.float32)
        y = y + b_ref[...]
        if slope is not None:
            y = jnp.where(y >= 0, y, slope * y)
        o_ref[...] = y

    return pl.pallas_call(
        body,
        grid=(_N // bn,),
        in_specs=[
            pl.BlockSpec((bn, _D), lambda i: (i, 0)),
            pl.BlockSpec((_NC, bn, _D), lambda i: (0, i, 0)),
            pl.BlockSpec((_D, _D), lambda i: (0, 0)),
            pl.BlockSpec((1, _D), lambda i: (0, 0)),
        ],
        out_specs=pl.BlockSpec((bn, _D), lambda i: (i, 0)),
        out_shape=jax.ShapeDtypeStruct((_N, _D), jnp.float32),
    )(x, acc, W, b.reshape(1, _D))


def _tc_layer_fc(x, acc, W1, b1, W2, b2):
    bn = 2000

    def body(x_ref, a_ref, w1_ref, b1_ref, w2_ref, b2_ref, o_ref):
        t = x_ref[...] + a_ref[0] + a_ref[1]
        y = jnp.dot(t, w1_ref[...], preferred_element_type=jnp.float32)
        y = y + b1_ref[...]
        y = jnp.dot(y, w2_ref[...], preferred_element_type=jnp.float32)
        o_ref[...] = y + b2_ref[...]

    return pl.pallas_call(
        body,
        grid=(_N // bn,),
        in_specs=[
            pl.BlockSpec((bn, _D), lambda i: (i, 0)),
            pl.BlockSpec((_NC, bn, _D), lambda i: (0, i, 0)),
            pl.BlockSpec((_D, _D), lambda i: (0, 0)),
            pl.BlockSpec((1, _D), lambda i: (0, 0)),
            pl.BlockSpec((_D, _D), lambda i: (0, 0)),
            pl.BlockSpec((1, _D), lambda i: (0, 0)),
        ],
        out_specs=pl.BlockSpec((bn, _D), lambda i: (i, 0)),
        out_shape=jax.ShapeDtypeStruct((_N, _D), jnp.float32),
    )(x, acc, W1, b1.reshape(1, _D), W2, b2.reshape(1, _D))


def kernel(n_feat_geo, nfeat_act, efeat_act, edge_index,
           W_geo1, b_geo1, W_geo2, b_geo2,
           W_act1, b_act1, W_act2, b_act2, W_fc, b_fc):
    src3 = edge_index[0].reshape(_NW, _NCHUNK, _CH)
    dst3 = edge_index[1].reshape(_NW, _NCHUNK, _CH)
    zrows = jnp.zeros((_RPT, _D), jnp.float32)

    agg = _sc_round(n_feat_geo, None, src3, dst3, zrows)
    h2 = _tc_layer(n_feat_geo, agg, W_geo1, b_geo1, 0.01)
    agg = _sc_round(h2, None, src3, dst3, zrows)
    h2 = _tc_layer(h2, agg, W_geo2, b_geo2, 0.01)

    agg = _sc_round(nfeat_act, efeat_act, src3, dst3, zrows)
    h1 = _tc_layer_fc(nfeat_act, agg, W_act1, b_act1, W_fc, b_fc)
    agg = _sc_round(h1, efeat_act, src3, dst3, zrows)
    h1 = _tc_layer(h1, agg, W_act2, b_act2, None)

    return jnp.concatenate([h1, h2], axis=1)


# speedup vs baseline: 6.1049x; 1.9154x over previous
import functools

import jax
import jax.numpy as jnp
from jax import lax
from jax.experimental import pallas as pl
from jax.experimental.pallas import tpu as pltpu
from jax.experimental.pallas import tpu_sc as plsc

_N = 10000
_E = 320000
_D = 128
_NC = 2
_NS = 16
_NW = _NC * _NS
_EPW = _E // _NW
_CH = 80
_NCHUNK = _EPW // _CH
_NPAD = 10112
_RPT = _NPAD // _NS
_VPR = _D // 16


def _sc_round(x, e, idx4, zrows):
    with_e = e is not None
    mesh = plsc.VectorSubcoreMesh(
        core_axis_name="c", subcore_axis_name="s",
        num_cores=_NC, num_subcores=_NS)

    scratch = [
        pltpu.VMEM((2, _CH), jnp.int32),
        pltpu.VMEM((2, _CH), jnp.int32),
        pltpu.VMEM((_CH, _D), jnp.float32),
        pltpu.VMEM((_CH, _D), jnp.float32),
        pltpu.VMEM_SHARED((_NPAD, _D), jnp.float32),
        pltpu.SemaphoreType.DMA,
        pltpu.SemaphoreType.DMA,
    ]
    if with_e:
        scratch[4:4] = [pltpu.VMEM((_CH, _D), jnp.float32),
                        pltpu.VMEM((_CH, _D), jnp.float32)]
        scratch += [pltpu.SemaphoreType.DMA, pltpu.SemaphoreType.DMA]

    def body(*refs):
        if with_e:
            (x_hbm, e_hbm, idx_hbm, z_hbm, out_hbm,
             ib0, ib1, rb0, rb1, eb0, eb1, acc,
             gs0, gs1, es0, es1) = refs
            ebuf = (eb0, eb1)
            esem = (es0, es1)
        else:
            (x_hbm, idx_hbm, z_hbm, out_hbm,
             ib0, ib1, rb0, rb1, acc, gs0, gs1) = refs
            e_hbm = ebuf = esem = None
        ibuf = (ib0, ib1)
        rbuf = (rb0, rb1)
        gsem = (gs0, gs1)
        c = lax.axis_index("c")
        s = lax.axis_index("s")
        wid = c * _NS + s

        pltpu.sync_copy(z_hbm, acc.at[pl.ds(s * _RPT, _RPT)])
        plsc.subcore_barrier()

        def start(j, p):
            pltpu.sync_copy(idx_hbm.at[wid, j], ibuf[p])
            pltpu.async_copy(x_hbm.at[ibuf[p].at[0]], rbuf[p], gsem[p])
            if with_e:
                off = wid * _EPW + j * _CH
                pltpu.async_copy(e_hbm.at[pl.ds(off, _CH)], ebuf[p], esem[p])

        def finish(j, p):
            pltpu.make_async_copy(
                x_hbm.at[ibuf[p].at[0]], rbuf[p], gsem[p]).wait()
            if with_e:
                off = wid * _EPW + j * _CH
                pltpu.make_async_copy(
                    e_hbm.at[pl.ds(off, _CH)], ebuf[p], esem[p]).wait()

                def rloop(i, c2):
                    for jj in range(_VPR):
                        sl = pl.ds(jj * 16, 16)
                        v = rbuf[p][i, sl] + ebuf[p][i, sl]
                        rbuf[p][i, sl] = jnp.maximum(v, 0.0)
                    return c2
                lax.fori_loop(0, _CH, rloop, 0)
            pltpu.sync_copy(rbuf[p], acc.at[ibuf[p].at[1]], add=True)

        start(0, 0)

        def pair(k, carry):
            for q in (0, 1):
                j = 2 * k + q
                start(j + 1, 1 - q)
                finish(j, q)
            return carry
        lax.fori_loop(0, (_NCHUNK - 1) // 2, pair, 0)
        finish(_NCHUNK - 1, (_NCHUNK - 1) % 2)
        plsc.subcore_barrier()

        pltpu.sync_copy(acc.at[pl.ds(s * _RPT, _RPT)],
                        out_hbm.at[c, pl.ds(s * _RPT, _RPT)])

    run = pl.kernel(
        body,
        out_type=jax.ShapeDtypeStruct((_NC, _NPAD, _D), jnp.float32),
        mesh=mesh,
        scratch_types=scratch,
    )
    if with_e:
        return run(x, e, idx4, zrows)
    return run(x, idx4, zrows)


def _tc_layer(x, acc, W, b, slope):
    bn = 2000

    def body(x_ref, a_ref, w_ref, b_ref, o_ref):
        t = x_ref[...] + a_ref[0] + a_ref[1]
        y = jnp.dot(t, w_ref[...], preferred_element_type=jnp.float32)
        y = y + b_ref[...]
        if slope is not None:
            y = jnp.where(y >= 0, y, slope * y)
        o_ref[...] = y

    return pl.pallas_call(
        body,
        grid=(_N // bn,),
        in_specs=[
            pl.BlockSpec((bn, _D), lambda i: (i, 0)),
            pl.BlockSpec((_NC, bn, _D), lambda i: (0, i, 0)),
            pl.BlockSpec((_D, _D), lambda i: (0, 0)),
            pl.BlockSpec((1, _D), lambda i: (0, 0)),
        ],
        out_specs=pl.BlockSpec((bn, _D), lambda i: (i, 0)),
        out_shape=jax.ShapeDtypeStruct((_N, _D), jnp.float32),
    )(x, acc, W, b.reshape(1, _D))


def _tc_layer_fc(x, acc, W1, b1, W2, b2):
    bn = 2000

    def body(x_ref, a_ref, w1_ref, b1_ref, w2_ref, b2_ref, o_ref):
        t = x_ref[...] + a_ref[0] + a_ref[1]
        y = jnp.dot(t, w1_ref[...], preferred_element_type=jnp.float32)
        y = y + b1_ref[...]
        y = jnp.dot(y, w2_ref[...], preferred_element_type=jnp.float32)
        o_ref[...] = y + b2_ref[...]

    return pl.pallas_call(
        body,
        grid=(_N // bn,),
        in_specs=[
            pl.BlockSpec((bn, _D), lambda i: (i, 0)),
            pl.BlockSpec((_NC, bn, _D), lambda i: (0, i, 0)),
            pl.BlockSpec((_D, _D), lambda i: (0, 0)),
            pl.BlockSpec((1, _D), lambda i: (0, 0)),
            pl.BlockSpec((_D, _D), lambda i: (0, 0)),
            pl.BlockSpec((1, _D), lambda i: (0, 0)),
        ],
        out_specs=pl.BlockSpec((bn, _D), lambda i: (i, 0)),
        out_shape=jax.ShapeDtypeStruct((_N, _D), jnp.float32),
    )(x, acc, W1, b1.reshape(1, _D), W2, b2.reshape(1, _D))


def kernel(n_feat_geo, nfeat_act, efeat_act, edge_index,
           W_geo1, b_geo1, W_geo2, b_geo2,
           W_act1, b_act1, W_act2, b_act2, W_fc, b_fc):
    idx4 = jnp.stack(
        [edge_index[0].reshape(_NW, _NCHUNK, _CH),
         edge_index[1].reshape(_NW, _NCHUNK, _CH)], axis=2)
    zrows = jnp.zeros((_RPT, _D), jnp.float32)

    agg = _sc_round(n_feat_geo, None, idx4, zrows)
    h2 = _tc_layer(n_feat_geo, agg, W_geo1, b_geo1, 0.01)
    agg = _sc_round(h2, None, idx4, zrows)
    h2 = _tc_layer(h2, agg, W_geo2, b_geo2, 0.01)

    agg = _sc_round(nfeat_act, efeat_act, idx4, zrows)
    h1 = _tc_layer_fc(nfeat_act, agg, W_act1, b_act1, W_fc, b_fc)
    agg = _sc_round(h1, efeat_act, idx4, zrows)
    h1 = _tc_layer(h1, agg, W_act2, b_act2, None)

    return jnp.concatenate([h1, h2], axis=1)


# speedup vs baseline: 7.3167x; 1.1985x over previous
import functools

import jax
import jax.numpy as jnp
from jax import lax
from jax.experimental import pallas as pl
from jax.experimental.pallas import tpu as pltpu
from jax.experimental.pallas import tpu_sc as plsc

_N = 10000
_E = 320000
_D = 128
_NC = 2
_NS = 16
_NW = _NC * _NS
_EPW = _E // _NW
_CH = 80
_NCHUNK = _EPW // _CH
_NPAD = 10112
_RPT = _NPAD // _NS
_VPR = _D // 16


def _sc_round(x, e, idx4, zrows):
    with_e = e is not None
    mesh = plsc.VectorSubcoreMesh(
        core_axis_name="c", subcore_axis_name="s",
        num_cores=_NC, num_subcores=_NS)

    scratch = [
        [pltpu.VMEM((2, _CH), jnp.int32)] * 3,
        [pltpu.VMEM((_CH, _D), jnp.float32)] * 2,
        pltpu.VMEM_SHARED((_NPAD, _D), jnp.float32),
        [pltpu.SemaphoreType.DMA] * 3,
        [pltpu.SemaphoreType.DMA] * 2,
    ]
    if with_e:
        scratch.insert(2, [pltpu.VMEM((_CH, _D), jnp.float32)] * 2)
        scratch.append([pltpu.SemaphoreType.DMA] * 2)

    def body(*refs):
        if with_e:
            (x_hbm, e_hbm, idx_hbm, z_hbm, out_hbm,
             ibuf, rbuf, ebuf, acc, isem, gsem, esem) = refs
        else:
            (x_hbm, idx_hbm, z_hbm, out_hbm,
             ibuf, rbuf, acc, isem, gsem) = refs
            e_hbm = ebuf = esem = None
        c = lax.axis_index("c")
        s = lax.axis_index("s")
        wid = c * _NS + s

        pltpu.sync_copy(z_hbm, acc.at[pl.ds(s * _RPT, _RPT)])
        plsc.subcore_barrier()

        def start_idx(j, q):
            pltpu.async_copy(idx_hbm.at[wid, j], ibuf[q % 3], isem[q % 3])

        def wait_idx(q):
            pltpu.make_async_copy(
                idx_hbm.at[wid, 0], ibuf[q % 3], isem[q % 3]).wait()

        def start_gather(j, q):
            pltpu.async_copy(
                x_hbm.at[ibuf[q % 3].at[0]], rbuf[q % 2], gsem[q % 2])
            if with_e:
                off = wid * _EPW + j * _CH
                pltpu.async_copy(
                    e_hbm.at[pl.ds(off, _CH)], ebuf[q % 2], esem[q % 2])

        def step(j, q, start_next=True, start_idx2=True):
            if start_next:
                wait_idx(q + 1)
                start_gather(j + 1, q + 1)
            if start_idx2:
                start_idx(j + 2, q + 2)
            pltpu.make_async_copy(
                x_hbm.at[ibuf[q % 3].at[0]], rbuf[q % 2],
                gsem[q % 2]).wait()
            if with_e:
                pltpu.make_async_copy(
                    e_hbm.at[pl.ds(0, _CH)], ebuf[q % 2],
                    esem[q % 2]).wait()

                def rloop(i, c2):
                    for jj in range(_VPR):
                        sl = pl.ds(jj * 16, 16)
                        v = rbuf[q % 2][i, sl] + ebuf[q % 2][i, sl]
                        rbuf[q % 2][i, sl] = jnp.maximum(v, 0.0)
                    return c2
                lax.fori_loop(0, _CH, rloop, 0)
            pltpu.sync_copy(rbuf[q % 2], acc.at[ibuf[q % 3].at[1]],
                            add=True)

        pltpu.sync_copy(idx_hbm.at[wid, 0], ibuf[0])
        start_gather(0, 0)
        start_idx(1, 1)
        step(0, 0)
        step(1, 1)

        def group(k, carry):
            j0 = 6 * k + 2
            for q in range(6):
                step(j0 + q, (2 + q) % 6)
            return carry
        lax.fori_loop(0, 20, group, 0)

        step(122, 2)
        step(123, 3, start_idx2=False)
        step(124, 4, start_next=False, start_idx2=False)
        plsc.subcore_barrier()

        pltpu.sync_copy(acc.at[pl.ds(s * _RPT, _RPT)],
                        out_hbm.at[c, pl.ds(s * _RPT, _RPT)])

    run = pl.kernel(
        body,
        out_type=jax.ShapeDtypeStruct((_NC, _NPAD, _D), jnp.float32),
        mesh=mesh,
        scratch_types=scratch,
    )
    if with_e:
        return run(x, e, idx4, zrows)
    return run(x, idx4, zrows)


def _tc_layer(x, acc, W, b, slope):
    bn = 2000

    def body(x_ref, a_ref, w_ref, b_ref, o_ref):
        t = x_ref[...] + a_ref[0] + a_ref[1]
        y = jnp.dot(t, w_ref[...], preferred_element_type=jnp.float32)
        y = y + b_ref[...]
        if slope is not None:
            y = jnp.where(y >= 0, y, slope * y)
        o_ref[...] = y

    return pl.pallas_call(
        body,
        grid=(_N // bn,),
        in_specs=[
            pl.BlockSpec((bn, _D), lambda i: (i, 0)),
            pl.BlockSpec((_NC, bn, _D), lambda i: (0, i, 0)),
            pl.BlockSpec((_D, _D), lambda i: (0, 0)),
            pl.BlockSpec((1, _D), lambda i: (0, 0)),
        ],
        out_specs=pl.BlockSpec((bn, _D), lambda i: (i, 0)),
        out_shape=jax.ShapeDtypeStruct((_N, _D), jnp.float32),
    )(x, acc, W, b.reshape(1, _D))


def _tc_layer_fc(x, acc, W1, b1, W2, b2):
    bn = 2000

    def body(x_ref, a_ref, w1_ref, b1_ref, w2_ref, b2_ref, o_ref):
        t = x_ref[...] + a_ref[0] + a_ref[1]
        y = jnp.dot(t, w1_ref[...], preferred_element_type=jnp.float32)
        y = y + b1_ref[...]
        y = jnp.dot(y, w2_ref[...], preferred_element_type=jnp.float32)
        o_ref[...] = y + b2_ref[...]

    return pl.pallas_call(
        body,
        grid=(_N // bn,),
        in_specs=[
            pl.BlockSpec((bn, _D), lambda i: (i, 0)),
            pl.BlockSpec((_NC, bn, _D), lambda i: (0, i, 0)),
            pl.BlockSpec((_D, _D), lambda i: (0, 0)),
            pl.BlockSpec((1, _D), lambda i: (0, 0)),
            pl.BlockSpec((_D, _D), lambda i: (0, 0)),
            pl.BlockSpec((1, _D), lambda i: (0, 0)),
        ],
        out_specs=pl.BlockSpec((bn, _D), lambda i: (i, 0)),
        out_shape=jax.ShapeDtypeStruct((_N, _D), jnp.float32),
    )(x, acc, W1, b1.reshape(1, _D), W2, b2.reshape(1, _D))


def kernel(n_feat_geo, nfeat_act, efeat_act, edge_index,
           W_geo1, b_geo1, W_geo2, b_geo2,
           W_act1, b_act1, W_act2, b_act2, W_fc, b_fc):
    idx4 = jnp.stack(
        [edge_index[0].reshape(_NW, _NCHUNK, _CH),
         edge_index[1].reshape(_NW, _NCHUNK, _CH)], axis=2)
    zrows = jnp.zeros((_RPT, _D), jnp.float32)

    agg = _sc_round(n_feat_geo, None, idx4, zrows)
    h2 = _tc_layer(n_feat_geo, agg, W_geo1, b_geo1, 0.01)
    agg = _sc_round(h2, None, idx4, zrows)
    h2 = _tc_layer(h2, agg, W_geo2, b_geo2, 0.01)

    agg = _sc_round(nfeat_act, efeat_act, idx4, zrows)
    h1 = _tc_layer_fc(nfeat_act, agg, W_act1, b_act1, W_fc, b_fc)
    agg = _sc_round(h1, efeat_act, idx4, zrows)
    h1 = _tc_layer(h1, agg, W_act2, b_act2, None)

    return jnp.concatenate([h1, h2], axis=1)
